# baseline (speedup 1.0000x reference)
import functools
import jax
import jax.numpy as jnp
from jax import lax
from jax.experimental import pallas as pl
from jax.experimental.pallas import tpu as pltpu
from jax.experimental.pallas import tpu_sc as plsc

_IN = 128
_HID = 64
_HEADS = 8
_OUT = 7
_N = 10000
_NV = 10240
_E = 330000
_NT = 32
_NB = 313
_NBP = 314
_CH = 2048
_NCHIN = (_E + _CH - 1) // _CH
_EPAD = _NCHIN * _CH
_CAP = _EPAD + _CH
_CHB = 1024
_CHB2 = 512
_DN1 = _NBP * _HEADS
_DN2 = 320


def _wid():
    return lax.axis_index("s") * 2 + lax.axis_index("c")


def _zero_f32(ref, n):
    def b(i, _):
        ref[pl.ds(pl.multiple_of(i * 16, 16), 16)] = (
            jnp.zeros((16,), jnp.float32))
        return 0
    lax.fori_loop(0, n // 16, b, 0)


def _bin_body(srcp, dstp, srcb, dstlb, cnts, sv, dv, stg_s, stg_d, cntv):
    wid = _wid()
    base = wid * _NB
    nb = jnp.minimum(_NB, _N - base)
    iota = lax.iota(jnp.int32, 16)

    def chunk(ci, total):
        pltpu.sync_copy(srcp.at[pl.ds(ci * _CH, _CH)], sv)
        pltpu.sync_copy(dstp.at[pl.ds(ci * _CH, _CH)], dv)

        def vec(j, cnt):
            j16 = pl.multiple_of(j * 16, 16)
            dvec = dv[pl.ds(j16, 16)]
            svec = sv[pl.ds(j16, 16)]
            dl = dvec - base
            m = (dl >= 0) & (dl < nb)
            run = plsc.cumsum(jnp.where(m, 1, 0).astype(jnp.int32))
            pos = cnt + run - 1
            plsc.store_scatter(stg_d, [pos], dl, mask=m)
            plsc.store_scatter(stg_s, [pos], svec, mask=m)
            return cnt + run[15]

        cnt = lax.fori_loop(0, _CH // 16, vec, jnp.int32(0))
        plsc.store_scatter(stg_d, [cnt + iota],
                           jnp.full((16,), _NB, jnp.int32))
        plsc.store_scatter(stg_s, [cnt + iota], jnp.zeros((16,), jnp.int32))
        cnt16 = jnp.bitwise_and(cnt + 15, -16)
        foff = pl.multiple_of(wid * _CAP + total, 16)
        pltpu.sync_copy(stg_s.at[pl.ds(0, _CH)],
                        srcb.at[pl.ds(foff, _CH)])
        pltpu.sync_copy(stg_d.at[pl.ds(0, _CH)],
                        dstlb.at[pl.ds(foff, _CH)])
        return total + cnt16

    total = lax.fori_loop(0, _NCHIN, chunk, jnp.int32(0))

    def fill(k, _):
        k16 = pl.multiple_of(k * 16, 16)
        stg_d[pl.ds(k16, 16)] = jnp.full((16,), _NB, jnp.int32)
        stg_s[pl.ds(k16, 16)] = jnp.zeros((16,), jnp.int32)
        return 0
    lax.fori_loop(0, _CH // 16, fill, 0)
    foff = pl.multiple_of(wid * _CAP + total, 16)
    pltpu.sync_copy(stg_s.at[pl.ds(0, _CH)], srcb.at[pl.ds(foff, _CH)])
    pltpu.sync_copy(stg_d.at[pl.ds(0, _CH)], dstlb.at[pl.ds(foff, _CH)])
    cntv[...] = jnp.broadcast_to(total, (16,))
    pltpu.sync_copy(cntv, cnts.at[pl.ds(pl.multiple_of(wid * 16, 16), 16)])


def _bin_edges(srcp, dstp):
    mesh = plsc.VectorSubcoreMesh(core_axis_name="c", subcore_axis_name="s")
    f = pl.kernel(
        _bin_body,
        out_type=[
            jax.ShapeDtypeStruct((_NT * _CAP,), jnp.int32),
            jax.ShapeDtypeStruct((_NT * _CAP,), jnp.int32),
            jax.ShapeDtypeStruct((_NT * 16,), jnp.int32),
        ],
        mesh=mesh,
        scratch_types=[
            pltpu.VMEM((_CH,), jnp.int32),
            pltpu.VMEM((_CH,), jnp.int32),
            pltpu.VMEM((_CH + 16,), jnp.int32),
            pltpu.VMEM((_CH + 16,), jnp.int32),
            pltpu.VMEM((16,), jnp.int32),
        ],
        compiler_params=pltpu.CompilerParams(needs_layout_passes=False),
        name="gat_bin",
    )
    return f(srcp, dstp)


def _make_att(nh, dn):
    tbl = _NV * nh
    if nh == 8:
        aload = 2512
        asz = 2512
    else:
        aload = 320
        asz = 336

    def body(srcb, dstlb, cnts, asrc_h, adst_h, exb, den_h,
             asrc_v, adst_v, den_v, sv, dl, exv, cntv):
        wid = _wid()
        base = wid * _NB
        iota = lax.iota(jnp.int32, 16)
        pltpu.sync_copy(cnts.at[pl.ds(pl.multiple_of(wid * 16, 16), 16)], cntv)
        count = cntv[...][0]
        pltpu.sync_copy(asrc_h.at[pl.ds(0, tbl)], asrc_v)
        _zero_f32(adst_v, asz)
        _zero_f32(den_v, dn)
        if nh == 8:
            off = pl.multiple_of(base * 8, 8)
            sh = jnp.int32(0)
        else:
            off = pl.multiple_of(jnp.bitwise_and(base, -8), 8)
            sh = base - off
        pltpu.sync_copy(adst_h.at[pl.ds(off, aload)],
                        adst_v.at[pl.ds(0, aload)])

        ntrip = (count + _CHB - 1) // _CHB

        def chunk(i, _):
            eoff = pl.multiple_of(wid * _CAP + i * _CHB, 16)
            pltpu.sync_copy(srcb.at[pl.ds(eoff, _CHB)], sv)
            pltpu.sync_copy(dstlb.at[pl.ds(eoff, _CHB)], dl)
            rem = jnp.minimum(_CHB, count - i * _CHB)
            nvec = rem * nh // 16

            def vec(j, _):
                j16 = pl.multiple_of(j * 16, 16)
                if nh == 8:
                    eidx = j * 2 + jnp.right_shift(iota, 3)
                    svals = plsc.load_gather(sv, [eidx])
                    dls = plsc.load_gather(dl, [eidx])
                    hh = jnp.bitwise_and(iota, 7)
                    a_s = plsc.load_gather(asrc_v, [svals * 8 + hh])
                    didx = dls * 8 + hh
                    a_d = plsc.load_gather(adst_v, [didx])
                    den_idx = didx
                else:
                    svals = sv[pl.ds(j16, 16)]
                    dls = dl[pl.ds(j16, 16)]
                    a_s = plsc.load_gather(asrc_v, [svals])
                    a_d = plsc.load_gather(adst_v, [dls + sh])
                    den_idx = dls
                al = a_s + a_d
                al = jnp.where(al < 0, al * jnp.float32(0.2), al)
                ex = jnp.exp(al)
                exv[pl.ds(j16, 16)] = ex
                plsc.addupdate_scatter(den_v, [den_idx], ex)
                return 0

            lax.fori_loop(0, nvec, vec, 0)
            pltpu.sync_copy(exv.at[pl.ds(0, _CHB * nh)],
                            exb.at[pl.ds(pl.multiple_of(eoff * nh, 8),
                                         _CHB * nh)])
            return 0

        lax.fori_loop(0, ntrip, chunk, 0)
        pltpu.sync_copy(den_v, den_h.at[pl.ds(pl.multiple_of(wid * dn, 8), dn)])

    def run(srcb, dstlb, cnts, asrc_flat, adst_flat):
        mesh = plsc.VectorSubcoreMesh(core_axis_name="c",
                                      subcore_axis_name="s")
        f = pl.kernel(
            body,
            out_type=[
                jax.ShapeDtypeStruct((_NT * _CAP * nh,), jnp.float32),
                jax.ShapeDtypeStruct((_NT * dn,), jnp.float32),
            ],
            mesh=mesh,
            scratch_types=[
                pltpu.VMEM((tbl,), jnp.float32),
                pltpu.VMEM((asz,), jnp.float32),
                pltpu.VMEM((dn,), jnp.float32),
                pltpu.VMEM((_CHB,), jnp.int32),
                pltpu.VMEM((_CHB,), jnp.int32),
                pltpu.VMEM((_CHB * nh,), jnp.float32),
                pltpu.VMEM((16,), jnp.int32),
            ],
            compiler_params=pltpu.CompilerParams(needs_layout_passes=False),
            name=f"gat_att{nh}",
        )
        return f(srcb, dstlb, cnts, asrc_flat, adst_flat)

    return run


def _make_agg(nh, cc, ccols, dn, hid, packed):
    wcols = ccols // 2 if packed else ccols

    def body(srcb, dstlb, cnts, den_h, exb, h_h, out_h,
             den_v, acc, cntv, cfb,
             sv0, sidx0, dl0, exv0, hbuf0, sl0, sg0,
             sv1, sidx1, dl1, exv1, hbuf1, sl1, sg1):
        wid = _wid()
        base = wid * _NB
        sid = lax.axis_index("s")
        iota = lax.iota(jnp.int32, 16)
        pltpu.sync_copy(cnts.at[pl.ds(pl.multiple_of(wid * 16, 16), 16)],
                        cntv)
        count = cntv[...][0]
        pltpu.sync_copy(den_h.at[pl.ds(pl.multiple_of(wid * dn, 8), dn)],
                        den_v.at[pl.ds(0, dn)])
        ntrip = (count + _CHB2 - 1) // _CHB2

        def fire_linear(i, sv, dl, exv, sem):
            eoff = pl.multiple_of(wid * _CAP + i * _CHB2, 16)
            pltpu.async_copy(srcb.at[pl.ds(eoff, _CHB2)], sv, sem)
            pltpu.async_copy(dstlb.at[pl.ds(eoff, _CHB2)],
                             dl.at[pl.ds(0, _CHB2)], sem)
            pltpu.async_copy(exb.at[pl.ds(pl.multiple_of(eoff * nh, 8),
                                          _CHB2 * nh)],
                             exv.at[pl.ds(0, _CHB2 * nh)], sem)

        def wait_linear(sv, dl, exv, sem):
            pltpu.make_async_copy(srcb.at[pl.ds(0, _CHB2)], sv, sem).wait()
            pltpu.make_async_copy(dstlb.at[pl.ds(0, _CHB2)],
                                  dl.at[pl.ds(0, _CHB2)], sem).wait()
            pltpu.make_async_copy(exb.at[pl.ds(0, _CHB2 * nh)],
                                  exv.at[pl.ds(0, _CHB2 * nh)], sem).wait()

        def fire_gather(c, sv, sidx, hbuf, sem):
            def mkj(j, _):
                j16 = pl.multiple_of(j * 16, 16)
                sidx[pl.ds(j16, 16)] = (
                    sv[pl.ds(j16, 16)] + jnp.int32(c * _NV))
                return 0
            lax.fori_loop(0, _CHB2 // 16, mkj, 0)
            for b in range(_CHB2 // 128):
                pltpu.async_copy(
                    h_h.at[sidx.at[pl.ds(b * 128, 128)]],
                    hbuf.at[pl.ds(b * 128, 128), :], sem)

        def wait_gather(hbuf, sem):
            for b in range(_CHB2 // 128):
                pltpu.make_async_copy(
                    h_h.at[pl.ds(0, 128), :],
                    hbuf.at[pl.ds(b * 128, 128), :], sem).wait()

        def process(i, c, dl, exv, hbuf):
            rem = jnp.clip(count - i * _CHB2, 0, _CHB2)
            hpc = ccols // hid

            def mkcoef(v, _):
                v16 = pl.multiple_of(v * 16, 16)
                lane = v16 + iota
                if hpc == 2:
                    e_lane = jnp.right_shift(lane, 1)
                    hdv = jnp.int32(c * hpc) + jnp.bitwise_and(iota, 1)
                else:
                    e_lane = lane
                    hdv = jnp.int32(c * hpc) + iota * 0
                dlg = plsc.load_gather(dl, [e_lane])
                exg = plsc.load_gather(exv, [e_lane * nh + hdv])
                deng = plsc.load_gather(den_v, [dlg * nh + hdv])
                cfb[pl.ds(v16, 16)] = exg / (deng + jnp.float32(1e-16))
                return 0

            lax.fori_loop(0, rem * hpc // 16, mkcoef, 0)

            colv = [jnp.int32(k * 16) + iota for k in range(wcols // 16)]

            def edge(e, _):
                e16 = e + iota * 0
                dlev = plsc.load_gather(dl, [e16])
                dbase = dlev * ccols
                cbs = [plsc.load_gather(cfb, [e16 * hpc + j])
                       for j in range(hpc)]
                for k in range(wcols // 16):
                    seg = plsc.load_gather(hbuf, [e16, colv[k]])
                    if packed:
                        j = (k * 32) // hid
                        lo = plsc.bitcast(
                            jax.lax.shift_left(seg, 16), jnp.float32)
                        hi = plsc.bitcast(
                            jnp.bitwise_and(seg, jnp.int32(-65536)),
                            jnp.float32)
                        idx_lo = dbase + jnp.int32(k * 32) + 2 * iota
                        plsc.addupdate_scatter(acc, [idx_lo], cbs[j] * lo)
                        plsc.addupdate_scatter(acc, [idx_lo + 1],
                                               cbs[j] * hi)
                    else:
                        j = (k * 16) // hid
                        plsc.addupdate_scatter(acc, [dbase + colv[k]],
                                               cbs[j] * seg)
                return 0

            lax.fori_loop(0, rem, edge, 0)

        for c in range(cc):
            _zero_f32(acc, _NBP * ccols)
            fire_linear(0, sv0, dl0, exv0, sl0)
            wait_linear(sv0, dl0, exv0, sl0)
            fire_gather(c, sv0, sidx0, hbuf0, sg0)
            fire_linear(1, sv1, dl1, exv1, sl1)

            def pair(t, _):
                i0 = 2 * t
                wait_linear(sv1, dl1, exv1, sl1)
                fire_gather(c, sv1, sidx1, hbuf1, sg1)
                wait_gather(hbuf0, sg0)
                process(i0, c, dl0, exv0, hbuf0)
                fire_linear(i0 + 2, sv0, dl0, exv0, sl0)
                wait_linear(sv0, dl0, exv0, sl0)
                fire_gather(c, sv0, sidx0, hbuf0, sg0)
                wait_gather(hbuf1, sg1)
                process(i0 + 1, c, dl1, exv1, hbuf1)
                fire_linear(i0 + 3, sv1, dl1, exv1, sl1)
                return 0

            lax.fori_loop(0, (ntrip + 1) // 2, pair, 0)
            wait_gather(hbuf0, sg0)
            wait_linear(sv1, dl1, exv1, sl1)
            pltpu.sync_copy(
                acc.at[pl.ds(0, _NB * ccols)],
                out_h.at[pl.ds(pl.multiple_of(
                    c * _NV * ccols + base * ccols, 8), _NB * ccols)])

    def run(srcb, dstlb, cnts, den, exb, h_flat):
        mesh = plsc.VectorSubcoreMesh(core_axis_name="c",
                                      subcore_axis_name="s")
        slot = [
            pltpu.VMEM((_CHB2,), jnp.int32),
            pltpu.VMEM((_CHB2,), jnp.int32),
            pltpu.VMEM((_CHB2 + 16,), jnp.int32),
            pltpu.VMEM((_CHB2 * nh + 16,), jnp.float32),
            pltpu.VMEM((_CHB2, wcols),
                       jnp.int32 if packed else jnp.float32),
            pltpu.SemaphoreType.DMA,
            pltpu.SemaphoreType.DMA,
        ]
        f = pl.kernel(
            body,
            out_type=jax.ShapeDtypeStruct((cc * _NV * ccols,), jnp.float32),
            mesh=mesh,
            scratch_types=[
                pltpu.VMEM((dn + 16,), jnp.float32),
                pltpu.VMEM((_NBP * ccols,), jnp.float32),
                pltpu.VMEM((16,), jnp.int32),
                pltpu.VMEM((_CHB2 * (ccols // hid) + 16,), jnp.float32),
            ] + slot + slot,
            compiler_params=pltpu.CompilerParams(
                needs_layout_passes=False, use_tc_tiling_on_sc=False),
            name=f"gat_agg{nh}",
        )
        return f(srcb, dstlb, cnts, den, exb, h_flat)

    return run


_att1 = _make_att(8, _DN1)
_att2 = _make_att(1, _DN2)
_agg1 = _make_agg(8, 4, 128, _DN1, 64, True)
_agg2 = _make_agg(1, 1, 16, _DN2, 16, False)


def _mm1_kernel(x_ref, w_ref, asrc_ref, adst_ref, h_ref, al_ref):
    c = pl.program_id(0)
    h = jnp.dot(x_ref[...], w_ref[...], preferred_element_type=jnp.float32)
    h_ref[...] = h[None]
    blk = h.shape[0]
    h3 = h.reshape(blk, 2, _HID)
    ridx = lax.broadcasted_iota(jnp.int32, (_HEADS, _HID), 0)
    asr = jnp.stack([
        jnp.sum(jnp.where(ridx == 2 * c + j, asrc_ref[...], 0.0), axis=0)
        for j in range(2)])
    adr = jnp.stack([
        jnp.sum(jnp.where(ridx == 2 * c + j, adst_ref[...], 0.0), axis=0)
        for j in range(2)])
    al_s = jnp.sum(h3 * asr[None], axis=-1)
    al_d = jnp.sum(h3 * adr[None], axis=-1)
    al_ref[...] = jnp.concatenate(
        [al_s, al_d, jnp.zeros((blk, 4), jnp.float32)], axis=-1)[None]


def _mm1(xp, W1, a_src1, a_dst1):
    blk = 1024
    h, al = pl.pallas_call(
        _mm1_kernel,
        grid=(4, _NV // blk),
        in_specs=[
            pl.BlockSpec((blk, _IN), lambda c, i: (i, 0)),
            pl.BlockSpec((_IN, 128), lambda c, i: (0, c)),
            pl.BlockSpec((_HEADS, _HID), lambda c, i: (0, 0)),
            pl.BlockSpec((_HEADS, _HID), lambda c, i: (0, 0)),
        ],
        out_specs=[
            pl.BlockSpec((1, blk, 128), lambda c, i: (c, i, 0)),
            pl.BlockSpec((1, blk, 8), lambda c, i: (c, i, 0)),
        ],
        out_shape=[
            jax.ShapeDtypeStruct((4, _NV, 128), jnp.float32),
            jax.ShapeDtypeStruct((4, _NV, 8), jnp.float32),
        ],
    )(xp, W1, a_src1, a_dst1)
    return h, al


def _mm2_kernel(a_ref, b1_ref, w2_ref, as2_ref, ad2_ref, h2_ref, al2_ref):
    hcat = jnp.concatenate(
        [a_ref[0], a_ref[1], a_ref[2], a_ref[3]], axis=-1)
    hb = hcat + b1_ref[...]
    h2in = jnp.where(hb > 0, hb, jnp.exp(jnp.minimum(hb, 0)) - 1.0)
    h2 = jnp.dot(h2in, w2_ref[...], preferred_element_type=jnp.float32)
    h2_ref[...] = h2
    blk = h2.shape[0]
    al_s = jnp.sum(h2 * as2_ref[...], axis=-1)
    al_d = jnp.sum(h2 * ad2_ref[...], axis=-1)
    al2_ref[...] = jnp.concatenate(
        [al_s[:, None], al_d[:, None], jnp.zeros((blk, 14), jnp.float32)],
        axis=-1)


def _mm2(agg1, b1, W2p, as2p, ad2p):
    blk = 1024
    h2, al2 = pl.pallas_call(
        _mm2_kernel,
        grid=(_NV // blk,),
        in_specs=[
            pl.BlockSpec((4, blk, 128), lambda i: (0, i, 0)),
            pl.BlockSpec((1, _HEADS * _HID), lambda i: (0, 0)),
            pl.BlockSpec((_HEADS * _HID, 16), lambda i: (0, 0)),
            pl.BlockSpec((1, 16), lambda i: (0, 0)),
            pl.BlockSpec((1, 16), lambda i: (0, 0)),
        ],
        out_specs=[
            pl.BlockSpec((blk, 16), lambda i: (i, 0)),
            pl.BlockSpec((blk, 16), lambda i: (i, 0)),
        ],
        out_shape=[
            jax.ShapeDtypeStruct((_NV, 16), jnp.float32),
            jax.ShapeDtypeStruct((_NV, 16), jnp.float32),
        ],
    )(agg1, b1, W2p, as2p, ad2p)
    return h2, al2


def kernel(x, edge_index, W1, a_src1, a_dst1, b1, W2, a_src2, a_dst2, b2):
    n = _N
    loop = jnp.arange(n, dtype=edge_index.dtype)
    pad = _EPAD - _E
    srcp = jnp.concatenate(
        [edge_index[0], loop, jnp.zeros((pad,), jnp.int32)])
    dstp = jnp.concatenate(
        [edge_index[1], loop, jnp.full((pad,), _N + 15, jnp.int32)])

    xp = jnp.zeros((_NV, _IN), jnp.float32).at[:n].set(x)

    h1, al1 = _mm1(xp, W1, a_src1, a_dst1)
    srcb, dstlb, cnts = _bin_edges(srcp, dstp)

    asrc1 = al1[:, :, 0:2].transpose(1, 0, 2).reshape(_NV * _HEADS)
    adst1 = al1[:, :, 2:4].transpose(1, 0, 2).reshape(_NV * _HEADS)
    ex1, den1 = _att1(srcb, dstlb, cnts, asrc1, adst1)
    hq1 = jax.lax.bitcast_convert_type(
        h1.astype(jnp.bfloat16).reshape(4 * _NV, 64, 2), jnp.int32)
    agg1 = _agg1(srcb, dstlb, cnts, den1, ex1, hq1).reshape(4, _NV, 128)

    W2p = jnp.zeros((_HEADS * _HID, 16), jnp.float32).at[:, :_OUT].set(W2)
    as2p = jnp.zeros((1, 16), jnp.float32).at[:, :_OUT].set(a_src2)
    ad2p = jnp.zeros((1, 16), jnp.float32).at[:, :_OUT].set(a_dst2)
    h2, al2 = _mm2(agg1, b1.reshape(1, -1), W2p, as2p, ad2p)

    ex2, den2 = _att2(srcb, dstlb, cnts, al2[:, 0].ravel(),
                      al2[:, 1].ravel())
    out = _agg2(srcb, dstlb, cnts, den2, ex2, h2.reshape(_NV, 16))
    return out.reshape(_NV, 16)[:n, :_OUT] + b2


# speedup vs baseline: 1.3265x; 1.3265x over previous
import functools
import jax
import jax.numpy as jnp
from jax import lax
from jax.experimental import pallas as pl
from jax.experimental.pallas import tpu as pltpu
from jax.experimental.pallas import tpu_sc as plsc

_IN = 128
_HID = 64
_HEADS = 8
_OUT = 7
_N = 10000
_NV = 10240
_E = 330000
_NT = 32
_NB = 313
_NBP = 314
_CH = 2048
_NCHIN = (_E + _CH - 1) // _CH
_EPAD = _NCHIN * _CH
_CAP = _EPAD + _CH
_CHB = 1024
_CHB2 = 256
_DN1 = _NBP * _HEADS
_DN2 = 320


def _wid():
    return lax.axis_index("s") * 2 + lax.axis_index("c")


def _zero_f32(ref, n):
    def b(i, _):
        ref[pl.ds(pl.multiple_of(i * 16, 16), 16)] = (
            jnp.zeros((16,), jnp.float32))
        return 0
    lax.fori_loop(0, n // 16, b, 0)


def _bin_body(srcp, dstp, srcb, dstlb, cnts, sv, dv, stg_s, stg_d, cntv):
    wid = _wid()
    base = wid * _NB
    nb = jnp.minimum(_NB, _N - base)
    iota = lax.iota(jnp.int32, 16)

    def chunk(ci, total):
        pltpu.sync_copy(srcp.at[pl.ds(ci * _CH, _CH)], sv)
        pltpu.sync_copy(dstp.at[pl.ds(ci * _CH, _CH)], dv)

        def vec(j, cnt):
            j16 = pl.multiple_of(j * 16, 16)
            dvec = dv[pl.ds(j16, 16)]
            svec = sv[pl.ds(j16, 16)]
            dl = dvec - base
            m = (dl >= 0) & (dl < nb)
            run = plsc.cumsum(jnp.where(m, 1, 0).astype(jnp.int32))
            pos = cnt + run - 1
            plsc.store_scatter(stg_d, [pos], dl, mask=m)
            plsc.store_scatter(stg_s, [pos], svec, mask=m)
            return cnt + run[15]

        cnt = lax.fori_loop(0, _CH // 16, vec, jnp.int32(0))
        plsc.store_scatter(stg_d, [cnt + iota],
                           jnp.full((16,), _NB, jnp.int32))
        plsc.store_scatter(stg_s, [cnt + iota], jnp.zeros((16,), jnp.int32))
        cnt16 = jnp.bitwise_and(cnt + 15, -16)
        foff = pl.multiple_of(wid * _CAP + total, 16)
        pltpu.sync_copy(stg_s.at[pl.ds(0, _CH)],
                        srcb.at[pl.ds(foff, _CH)])
        pltpu.sync_copy(stg_d.at[pl.ds(0, _CH)],
                        dstlb.at[pl.ds(foff, _CH)])
        return total + cnt16

    total = lax.fori_loop(0, _NCHIN, chunk, jnp.int32(0))

    def fill(k, _):
        k16 = pl.multiple_of(k * 16, 16)
        stg_d[pl.ds(k16, 16)] = jnp.full((16,), _NB, jnp.int32)
        stg_s[pl.ds(k16, 16)] = jnp.zeros((16,), jnp.int32)
        return 0
    lax.fori_loop(0, _CH // 16, fill, 0)
    foff = pl.multiple_of(wid * _CAP + total, 16)
    pltpu.sync_copy(stg_s.at[pl.ds(0, _CH)], srcb.at[pl.ds(foff, _CH)])
    pltpu.sync_copy(stg_d.at[pl.ds(0, _CH)], dstlb.at[pl.ds(foff, _CH)])
    cntv[...] = jnp.broadcast_to(total, (16,))
    pltpu.sync_copy(cntv, cnts.at[pl.ds(pl.multiple_of(wid * 16, 16), 16)])


def _bin_edges(srcp, dstp):
    mesh = plsc.VectorSubcoreMesh(core_axis_name="c", subcore_axis_name="s")
    f = pl.kernel(
        _bin_body,
        out_type=[
            jax.ShapeDtypeStruct((_NT * _CAP,), jnp.int32),
            jax.ShapeDtypeStruct((_NT * _CAP,), jnp.int32),
            jax.ShapeDtypeStruct((_NT * 16,), jnp.int32),
        ],
        mesh=mesh,
        scratch_types=[
            pltpu.VMEM((_CH,), jnp.int32),
            pltpu.VMEM((_CH,), jnp.int32),
            pltpu.VMEM((_CH + 16,), jnp.int32),
            pltpu.VMEM((_CH + 16,), jnp.int32),
            pltpu.VMEM((16,), jnp.int32),
        ],
        compiler_params=pltpu.CompilerParams(needs_layout_passes=False),
        name="gat_bin",
    )
    return f(srcp, dstp)


def _make_att(nh, dn):
    tbl = _NV * nh
    if nh == 8:
        aload = 2512
        asz = 2512
    else:
        aload = 320
        asz = 336

    def body(srcb, dstlb, cnts, asrc_h, adst_h, exb, den_h,
             asrc_v, adst_v, den_v, sv, dl, exv, cntv):
        wid = _wid()
        base = wid * _NB
        iota = lax.iota(jnp.int32, 16)
        pltpu.sync_copy(cnts.at[pl.ds(pl.multiple_of(wid * 16, 16), 16)], cntv)
        count = cntv[...][0]
        pltpu.sync_copy(asrc_h.at[pl.ds(0, tbl)], asrc_v)
        _zero_f32(adst_v, asz)
        _zero_f32(den_v, dn)
        if nh == 8:
            off = pl.multiple_of(base * 8, 8)
            sh = jnp.int32(0)
        else:
            off = pl.multiple_of(jnp.bitwise_and(base, -8), 8)
            sh = base - off
        pltpu.sync_copy(adst_h.at[pl.ds(off, aload)],
                        adst_v.at[pl.ds(0, aload)])

        ntrip = (count + _CHB - 1) // _CHB

        def chunk(i, _):
            eoff = pl.multiple_of(wid * _CAP + i * _CHB, 16)
            pltpu.sync_copy(srcb.at[pl.ds(eoff, _CHB)], sv)
            pltpu.sync_copy(dstlb.at[pl.ds(eoff, _CHB)], dl)
            rem = jnp.minimum(_CHB, count - i * _CHB)
            nvec = rem * nh // 16

            def vec(j, _):
                j16 = pl.multiple_of(j * 16, 16)
                if nh == 8:
                    eidx = j * 2 + jnp.right_shift(iota, 3)
                    svals = plsc.load_gather(sv, [eidx])
                    dls = plsc.load_gather(dl, [eidx])
                    hh = jnp.bitwise_and(iota, 7)
                    a_s = plsc.load_gather(asrc_v, [svals * 8 + hh])
                    didx = dls * 8 + hh
                    a_d = plsc.load_gather(adst_v, [didx])
                    den_idx = didx
                else:
                    svals = sv[pl.ds(j16, 16)]
                    dls = dl[pl.ds(j16, 16)]
                    a_s = plsc.load_gather(asrc_v, [svals])
                    a_d = plsc.load_gather(adst_v, [dls + sh])
                    den_idx = dls
                al = a_s + a_d
                al = jnp.where(al < 0, al * jnp.float32(0.2), al)
                ex = jnp.exp(al)
                exv[pl.ds(j16, 16)] = ex
                plsc.addupdate_scatter(den_v, [den_idx], ex)
                return 0

            lax.fori_loop(0, nvec, vec, 0)
            pltpu.sync_copy(exv.at[pl.ds(0, _CHB * nh)],
                            exb.at[pl.ds(pl.multiple_of(eoff * nh, 8),
                                         _CHB * nh)])
            return 0

        lax.fori_loop(0, ntrip, chunk, 0)
        pltpu.sync_copy(den_v, den_h.at[pl.ds(pl.multiple_of(wid * dn, 8), dn)])

    def run(srcb, dstlb, cnts, asrc_flat, adst_flat):
        mesh = plsc.VectorSubcoreMesh(core_axis_name="c",
                                      subcore_axis_name="s")
        f = pl.kernel(
            body,
            out_type=[
                jax.ShapeDtypeStruct((_NT * _CAP * nh,), jnp.float32),
                jax.ShapeDtypeStruct((_NT * dn,), jnp.float32),
            ],
            mesh=mesh,
            scratch_types=[
                pltpu.VMEM((tbl,), jnp.float32),
                pltpu.VMEM((asz,), jnp.float32),
                pltpu.VMEM((dn,), jnp.float32),
                pltpu.VMEM((_CHB,), jnp.int32),
                pltpu.VMEM((_CHB,), jnp.int32),
                pltpu.VMEM((_CHB * nh,), jnp.float32),
                pltpu.VMEM((16,), jnp.int32),
            ],
            compiler_params=pltpu.CompilerParams(needs_layout_passes=False),
            name=f"gat_att{nh}",
        )
        return f(srcb, dstlb, cnts, asrc_flat, adst_flat)

    return run


def _make_agg(nh, cc, ccols, dn, hid, packed):
    wcols = ccols // 2 if packed else ccols

    def body(srcb, dstlb, cnts, den_h, exb, h_h, out_h,
             den_v, acc, cntv, cfb,
             sv0, sidx0, dl0, exv0, hbuf0, sl0, sg0,
             sv1, sidx1, dl1, exv1, hbuf1, sl1, sg1):
        wid = _wid()
        base = wid * _NB
        sid = lax.axis_index("s")
        iota = lax.iota(jnp.int32, 16)
        pltpu.sync_copy(cnts.at[pl.ds(pl.multiple_of(wid * 16, 16), 16)],
                        cntv)
        count = cntv[...][0]
        pltpu.sync_copy(den_h.at[pl.ds(pl.multiple_of(wid * dn, 8), dn)],
                        den_v.at[pl.ds(0, dn)])
        ntrip = (count + _CHB2 - 1) // _CHB2

        def fire_linear(i, sv, dl, exv, sem):
            eoff = pl.multiple_of(wid * _CAP + i * _CHB2, 16)
            pltpu.async_copy(srcb.at[pl.ds(eoff, _CHB2)], sv, sem)
            pltpu.async_copy(dstlb.at[pl.ds(eoff, _CHB2)],
                             dl.at[pl.ds(0, _CHB2)], sem)
            pltpu.async_copy(exb.at[pl.ds(pl.multiple_of(eoff * nh, 8),
                                          _CHB2 * nh)],
                             exv.at[pl.ds(0, _CHB2 * nh)], sem)

        def wait_linear(sv, dl, exv, sem):
            pltpu.make_async_copy(srcb.at[pl.ds(0, _CHB2)], sv, sem).wait()
            pltpu.make_async_copy(dstlb.at[pl.ds(0, _CHB2)],
                                  dl.at[pl.ds(0, _CHB2)], sem).wait()
            pltpu.make_async_copy(exb.at[pl.ds(0, _CHB2 * nh)],
                                  exv.at[pl.ds(0, _CHB2 * nh)], sem).wait()

        def fire_gather(c, sv, sidx, hbuf, sem):
            def mkj(j, _):
                j16 = pl.multiple_of(j * 16, 16)
                sidx[pl.ds(j16, 16)] = (
                    sv[pl.ds(j16, 16)] + jnp.int32(c * _NV))
                return 0
            lax.fori_loop(0, _CHB2 // 16, mkj, 0)
            for b in range(_CHB2 // 128):
                pltpu.async_copy(
                    h_h.at[sidx.at[pl.ds(b * 128, 128)]],
                    hbuf.at[pl.ds(b * 128, 128), :], sem)

        def wait_gather(hbuf, sem):
            for b in range(_CHB2 // 128):
                pltpu.make_async_copy(
                    h_h.at[pl.ds(0, 128), :],
                    hbuf.at[pl.ds(b * 128, 128), :], sem).wait()

        def process(i, c, dl, exv, hbuf):
            rem = jnp.clip(count - i * _CHB2, 0, _CHB2)
            hpc = ccols // hid

            def mkcoef(v, _):
                v16 = pl.multiple_of(v * 16, 16)
                lane = v16 + iota
                if hpc == 2:
                    e_lane = jnp.right_shift(lane, 1)
                    hdv = jnp.int32(c * hpc) + jnp.bitwise_and(iota, 1)
                else:
                    e_lane = lane
                    hdv = jnp.int32(c * hpc) + iota * 0
                dlg = plsc.load_gather(dl, [e_lane])
                exg = plsc.load_gather(exv, [e_lane * nh + hdv])
                deng = plsc.load_gather(den_v, [dlg * nh + hdv])
                cfb[pl.ds(v16, 16)] = exg / (deng + jnp.float32(1e-16))
                return 0

            lax.fori_loop(0, rem * hpc // 16, mkcoef, 0)

            colv = [jnp.int32(k * 16) + iota for k in range(wcols // 16)]

            def edge(e, _):
                e16 = e + iota * 0
                dlev = plsc.load_gather(dl, [e16])
                dbase = dlev * ccols
                cbs = [plsc.load_gather(cfb, [e16 * hpc + j])
                       for j in range(hpc)]
                for k in range(wcols // 16):
                    seg = plsc.load_gather(hbuf, [e16, colv[k]])
                    if packed:
                        j = (k * 32) // hid
                        lo = plsc.bitcast(
                            jax.lax.shift_left(seg, 16), jnp.float32)
                        hi = plsc.bitcast(
                            jnp.bitwise_and(seg, jnp.int32(-65536)),
                            jnp.float32)
                        idx_lo = dbase + jnp.int32(k * 32) + 2 * iota
                        plsc.addupdate_scatter(acc, [idx_lo], cbs[j] * lo)
                        plsc.addupdate_scatter(acc, [idx_lo + 1],
                                               cbs[j] * hi)
                    else:
                        j = (k * 16) // hid
                        plsc.addupdate_scatter(acc, [dbase + colv[k]],
                                               cbs[j] * seg)
                return 0

            lax.fori_loop(0, rem, edge, 0)

        for c in range(cc):
            _zero_f32(acc, _NBP * ccols)
            fire_linear(0, sv0, dl0, exv0, sl0)
            wait_linear(sv0, dl0, exv0, sl0)
            fire_gather(c, sv0, sidx0, hbuf0, sg0)
            fire_linear(1, sv1, dl1, exv1, sl1)

            def pair(t, _):
                i0 = 2 * t
                wait_linear(sv1, dl1, exv1, sl1)
                fire_gather(c, sv1, sidx1, hbuf1, sg1)
                wait_gather(hbuf0, sg0)
                process(i0, c, dl0, exv0, hbuf0)
                fire_linear(i0 + 2, sv0, dl0, exv0, sl0)
                wait_linear(sv0, dl0, exv0, sl0)
                fire_gather(c, sv0, sidx0, hbuf0, sg0)
                wait_gather(hbuf1, sg1)
                process(i0 + 1, c, dl1, exv1, hbuf1)
                fire_linear(i0 + 3, sv1, dl1, exv1, sl1)
                return 0

            lax.fori_loop(0, (ntrip + 1) // 2, pair, 0)
            wait_gather(hbuf0, sg0)
            wait_linear(sv1, dl1, exv1, sl1)
            pltpu.sync_copy(
                acc.at[pl.ds(0, _NB * ccols)],
                out_h.at[pl.ds(pl.multiple_of(
                    c * _NV * ccols + base * ccols, 8), _NB * ccols)])

    def run(srcb, dstlb, cnts, den, exb, h_flat):
        mesh = plsc.VectorSubcoreMesh(core_axis_name="c",
                                      subcore_axis_name="s")
        slot = [
            pltpu.VMEM((_CHB2,), jnp.int32),
            pltpu.VMEM((_CHB2,), jnp.int32),
            pltpu.VMEM((_CHB2 + 16,), jnp.int32),
            pltpu.VMEM((_CHB2 * nh + 16,), jnp.float32),
            pltpu.VMEM((_CHB2, wcols),
                       jnp.int32 if packed else jnp.float32),
            pltpu.SemaphoreType.DMA,
            pltpu.SemaphoreType.DMA,
        ]
        f = pl.kernel(
            body,
            out_type=jax.ShapeDtypeStruct((cc * _NV * ccols,), jnp.float32),
            mesh=mesh,
            scratch_types=[
                pltpu.VMEM((dn + 16,), jnp.float32),
                pltpu.VMEM((_NBP * ccols,), jnp.float32),
                pltpu.VMEM((16,), jnp.int32),
                pltpu.VMEM((_CHB2 * (ccols // hid) + 16,), jnp.float32),
            ] + slot + slot,
            compiler_params=pltpu.CompilerParams(
                needs_layout_passes=False, use_tc_tiling_on_sc=False),
            name=f"gat_agg{nh}",
        )
        return f(srcb, dstlb, cnts, den, exb, h_flat)

    return run


_att1 = _make_att(8, _DN1)
_att2 = _make_att(1, _DN2)
_agg1 = _make_agg(8, 4, 128, _DN1, 64, True)
_agg2 = _make_agg(1, 1, 16, _DN2, 16, False)


def _mm1_kernel(x_ref, w_ref, asrc_ref, adst_ref, h_ref, al_ref):
    c = pl.program_id(0)
    h = jnp.dot(x_ref[...], w_ref[...], preferred_element_type=jnp.float32)
    h_ref[...] = h[None]
    blk = h.shape[0]
    h3 = h.reshape(blk, 2, _HID)
    ridx = lax.broadcasted_iota(jnp.int32, (_HEADS, _HID), 0)
    asr = jnp.stack([
        jnp.sum(jnp.where(ridx == 2 * c + j, asrc_ref[...], 0.0), axis=0)
        for j in range(2)])
    adr = jnp.stack([
        jnp.sum(jnp.where(ridx == 2 * c + j, adst_ref[...], 0.0), axis=0)
        for j in range(2)])
    al_s = jnp.sum(h3 * asr[None], axis=-1)
    al_d = jnp.sum(h3 * adr[None], axis=-1)
    al_ref[...] = jnp.concatenate(
        [al_s, al_d, jnp.zeros((blk, 4), jnp.float32)], axis=-1)[None]


def _mm1(xp, W1, a_src1, a_dst1):
    blk = 1024
    h, al = pl.pallas_call(
        _mm1_kernel,
        grid=(4, _NV // blk),
        in_specs=[
            pl.BlockSpec((blk, _IN), lambda c, i: (i, 0)),
            pl.BlockSpec((_IN, 128), lambda c, i: (0, c)),
            pl.BlockSpec((_HEADS, _HID), lambda c, i: (0, 0)),
            pl.BlockSpec((_HEADS, _HID), lambda c, i: (0, 0)),
        ],
        out_specs=[
            pl.BlockSpec((1, blk, 128), lambda c, i: (c, i, 0)),
            pl.BlockSpec((1, blk, 8), lambda c, i: (c, i, 0)),
        ],
        out_shape=[
            jax.ShapeDtypeStruct((4, _NV, 128), jnp.float32),
            jax.ShapeDtypeStruct((4, _NV, 8), jnp.float32),
        ],
    )(xp, W1, a_src1, a_dst1)
    return h, al


def _mm2_kernel(a_ref, b1_ref, w2_ref, as2_ref, ad2_ref, h2_ref, al2_ref):
    hcat = jnp.concatenate(
        [a_ref[0], a_ref[1], a_ref[2], a_ref[3]], axis=-1)
    hb = hcat + b1_ref[...]
    h2in = jnp.where(hb > 0, hb, jnp.exp(jnp.minimum(hb, 0)) - 1.0)
    h2 = jnp.dot(h2in, w2_ref[...], preferred_element_type=jnp.float32)
    h2_ref[...] = h2
    blk = h2.shape[0]
    al_s = jnp.sum(h2 * as2_ref[...], axis=-1)
    al_d = jnp.sum(h2 * ad2_ref[...], axis=-1)
    al2_ref[...] = jnp.concatenate(
        [al_s[:, None], al_d[:, None], jnp.zeros((blk, 14), jnp.float32)],
        axis=-1)


def _mm2(agg1, b1, W2p, as2p, ad2p):
    blk = 1024
    h2, al2 = pl.pallas_call(
        _mm2_kernel,
        grid=(_NV // blk,),
        in_specs=[
            pl.BlockSpec((4, blk, 128), lambda i: (0, i, 0)),
            pl.BlockSpec((1, _HEADS * _HID), lambda i: (0, 0)),
            pl.BlockSpec((_HEADS * _HID, 16), lambda i: (0, 0)),
            pl.BlockSpec((1, 16), lambda i: (0, 0)),
            pl.BlockSpec((1, 16), lambda i: (0, 0)),
        ],
        out_specs=[
            pl.BlockSpec((blk, 16), lambda i: (i, 0)),
            pl.BlockSpec((blk, 16), lambda i: (i, 0)),
        ],
        out_shape=[
            jax.ShapeDtypeStruct((_NV, 16), jnp.float32),
            jax.ShapeDtypeStruct((_NV, 16), jnp.float32),
        ],
    )(agg1, b1, W2p, as2p, ad2p)
    return h2, al2


def kernel(x, edge_index, W1, a_src1, a_dst1, b1, W2, a_src2, a_dst2, b2):
    n = _N
    loop = jnp.arange(n, dtype=edge_index.dtype)
    pad = _EPAD - _E
    srcp = jnp.concatenate(
        [edge_index[0], loop, jnp.zeros((pad,), jnp.int32)])
    dstp = jnp.concatenate(
        [edge_index[1], loop, jnp.full((pad,), _N + 15, jnp.int32)])

    xp = jnp.zeros((_NV, _IN), jnp.float32).at[:n].set(x)

    h1, al1 = _mm1(xp, W1, a_src1, a_dst1)
    srcb, dstlb, cnts = _bin_edges(srcp, dstp)

    asrc1 = al1[:, :, 0:2].transpose(1, 0, 2).reshape(_NV * _HEADS)
    adst1 = al1[:, :, 2:4].transpose(1, 0, 2).reshape(_NV * _HEADS)
    ex1, den1 = _att1(srcb, dstlb, cnts, asrc1, adst1)
    hq1 = jax.lax.bitcast_convert_type(
        h1.astype(jnp.bfloat16).reshape(4 * _NV, 64, 2), jnp.int32)
    agg1 = _agg1(srcb, dstlb, cnts, den1, ex1, hq1).reshape(4, _NV, 128)

    W2p = jnp.zeros((_HEADS * _HID, 16), jnp.float32).at[:, :_OUT].set(W2)
    as2p = jnp.zeros((1, 16), jnp.float32).at[:, :_OUT].set(a_src2)
    ad2p = jnp.zeros((1, 16), jnp.float32).at[:, :_OUT].set(a_dst2)
    h2, al2 = _mm2(agg1, b1.reshape(1, -1), W2p, as2p, ad2p)

    ex2, den2 = _att2(srcb, dstlb, cnts, al2[:, 0].ravel(),
                      al2[:, 1].ravel())
    out = _agg2(srcb, dstlb, cnts, den2, ex2, h2.reshape(_NV, 16))
    return out.reshape(_NV, 16)[:n, :_OUT] + b2
